# baseline (speedup 1.0000x reference)
import functools

import jax
import jax.numpy as jnp
from jax import lax
from jax.experimental import pallas as pl
from jax.experimental.pallas import tpu as pltpu
from jax.experimental.pallas import tpu_sc as plsc

NC = 2
NS = 16
LANES = 16
NW = NC * NS

CH = 128
PR = 128
F = 32


def _gmf_body(pu_ref, po_ref, pi_ref, qo_ref, utab_ref, itab_ref, wb_ref,
              out_ref, idx_u, idx_i, off_u, off_i, buf_u, buf_i, wb_v,
              out_v, sem):
    n_chunks = idx_u.shape[0]
    bpw = n_chunks * CH

    wid = lax.axis_index("s") * NC + lax.axis_index("c")
    base = wid * bpw

    pltpu.sync_copy(pu_ref.at[wid], idx_u)
    pltpu.sync_copy(pi_ref.at[wid], idx_i)
    pltpu.sync_copy(po_ref.at[pl.ds(base, bpw)], off_u)
    pltpu.sync_copy(qo_ref.at[pl.ds(base, bpw)], off_i)
    pltpu.sync_copy(wb_ref, wb_v)

    w0 = wb_v[pl.ds(0, LANES)]
    w1 = wb_v[pl.ds(LANES, LANES)]
    bias = wb_v[pl.ds(2 * LANES, LANES)]
    iota16 = lax.iota(jnp.int32, LANES)

    def start(k):
        return (
            pltpu.async_copy(utab_ref.at[idx_u.at[k]], buf_u.at[k % 2], sem),
            pltpu.async_copy(itab_ref.at[idx_i.at[k]], buf_i.at[k % 2], sem),
        )

    pending = start(0)
    for k in range(n_chunks):
        nxt = start(k + 1) if k + 1 < n_chunks else None
        pending[0].wait()
        pending[1].wait()
        slot = k % 2

        def group(g, _):
            r0 = k * CH + g * LANES
            offu = off_u[pl.ds(r0, LANES)]
            offi = off_i[pl.ds(r0, LANES)]
            rr = g * LANES
            z = bias
            for j in range(LANES):
                ou = offu[j]
                oi = offi[j]
                u0 = buf_u[slot, rr + j, pl.ds(ou, LANES)]
                u1 = buf_u[slot, rr + j, pl.ds(ou + LANES, LANES)]
                v0 = buf_i[slot, rr + j, pl.ds(oi, LANES)]
                v1 = buf_i[slot, rr + j, pl.ds(oi + LANES, LANES)]
                s = jnp.sum(u0 * v0 * w0 + u1 * v1 * w1)
                z = jnp.where(iota16 == j, z + s, z)
            out_v[pl.ds(r0, LANES)] = 1.0 / (1.0 + jnp.exp(-z))
            return _

        lax.fori_loop(0, CH // LANES, group, None)
        pending = nxt

    pltpu.sync_copy(out_v, out_ref.at[pl.ds(base, bpw)])


def kernel(users, items, user_table, item_table, pred_w, pred_b):
    b = users.shape[0]
    n, f = user_table.shape
    assert b % (NW * CH) == 0 and f == F and (n * f) % PR == 0
    bpw = b // NW
    n_chunks = bpw // CH
    rows_per_pr = PR // F

    ut = user_table.reshape(-1, PR)
    it = item_table.reshape(-1, PR)
    ui = users.astype(jnp.int32)
    ii = items.astype(jnp.int32)
    pu = (ui // rows_per_pr).reshape(NW, n_chunks, CH)
    pi = (ii // rows_per_pr).reshape(NW, n_chunks, CH)
    po = (ui % rows_per_pr) * F
    qo = (ii % rows_per_pr) * F
    wb = jnp.concatenate(
        [pred_w.reshape(-1), jnp.broadcast_to(pred_b.reshape(-1)[:1], (LANES,))]
    ).astype(jnp.float32)

    mesh = plsc.VectorSubcoreMesh(core_axis_name="c", subcore_axis_name="s")
    run = functools.partial(
        pl.kernel,
        out_type=jax.ShapeDtypeStruct((b,), jnp.float32),
        mesh=mesh,
        compiler_params=pltpu.CompilerParams(
            needs_layout_passes=False, use_tc_tiling_on_sc=True
        ),
        scratch_types=[
            pltpu.VMEM((n_chunks, CH), jnp.int32),
            pltpu.VMEM((n_chunks, CH), jnp.int32),
            pltpu.VMEM((bpw,), jnp.int32),
            pltpu.VMEM((bpw,), jnp.int32),
            pltpu.VMEM((2, CH, PR), jnp.float32),
            pltpu.VMEM((2, CH, PR), jnp.float32),
            pltpu.VMEM((3 * LANES,), jnp.float32),
            pltpu.VMEM((bpw,), jnp.float32),
            pltpu.SemaphoreType.DMA,
        ],
    )(_gmf_body)
    return run(pu, po, pi, qo, ut, it, wb)


# speedup vs baseline: 1.0112x; 1.0112x over previous
import functools

import jax
import jax.numpy as jnp
from jax import lax
from jax.experimental import pallas as pl
from jax.experimental.pallas import tpu as pltpu
from jax.experimental.pallas import tpu_sc as plsc

NC = 2
NS = 16
LANES = 16
NW = NC * NS

IDX_CHUNK = 128


def _gmf_body(users_ref, items_ref, utab_ref, itab_ref, wb_ref, out_ref,
              idx_u, idx_i, rows_u, rows_i, wb_v, out_v, sem):
    bpw = rows_u.shape[0]
    n_chunks = bpw // IDX_CHUNK
    f = rows_u.shape[1]
    half = f // 2

    wid = lax.axis_index("s") * NC + lax.axis_index("c")
    base = wid * bpw

    pltpu.sync_copy(users_ref.at[wid], idx_u)
    pltpu.sync_copy(items_ref.at[wid], idx_i)
    pltpu.sync_copy(wb_ref, wb_v)

    copies = []
    for k in range(n_chunks):
        dst = rows_u.at[pl.ds(k * IDX_CHUNK, IDX_CHUNK)]
        copies.append(pltpu.async_copy(utab_ref.at[idx_u.at[k]], dst, sem))
    for k in range(n_chunks):
        dst = rows_i.at[pl.ds(k * IDX_CHUNK, IDX_CHUNK)]
        copies.append(pltpu.async_copy(itab_ref.at[idx_i.at[k]], dst, sem))
    for c in copies:
        c.wait()

    w0 = wb_v[pl.ds(0, LANES)]
    w1 = wb_v[pl.ds(half, LANES)]
    bias = wb_v[pl.ds(f, LANES)]
    iota16 = lax.iota(jnp.int32, LANES)

    def group(g, _):
        r0 = g * LANES
        z = bias
        for j in range(LANES):
            r = r0 + j
            u0 = rows_u[r, pl.ds(0, LANES)]
            u1 = rows_u[r, pl.ds(half, LANES)]
            v0 = rows_i[r, pl.ds(0, LANES)]
            v1 = rows_i[r, pl.ds(half, LANES)]
            s = jnp.sum(u0 * v0 * w0 + u1 * v1 * w1)
            z = jnp.where(iota16 == j, z + s, z)
        out_v[pl.ds(r0, LANES)] = 1.0 / (1.0 + jnp.exp(-z))
        return _

    lax.fori_loop(0, bpw // LANES, group, None)

    pltpu.sync_copy(out_v, out_ref.at[pl.ds(base, bpw)])


def _linearize(t):
    flat = lax.optimization_barrier(t.reshape(-1))
    return flat.reshape(t.shape)


def kernel(users, items, user_table, item_table, pred_w, pred_b):
    b = users.shape[0]
    f = user_table.shape[1]
    assert b % (NW * IDX_CHUNK) == 0 and f == 2 * LANES
    bpw = b // NW

    users_r = users.astype(jnp.int32).reshape(NW, bpw // IDX_CHUNK, IDX_CHUNK)
    items_r = items.astype(jnp.int32).reshape(NW, bpw // IDX_CHUNK, IDX_CHUNK)
    ut = _linearize(user_table)
    it = _linearize(item_table)
    wb = jnp.concatenate(
        [pred_w.reshape(-1), jnp.broadcast_to(pred_b.reshape(-1)[:1], (LANES,))]
    ).astype(jnp.float32)

    mesh = plsc.VectorSubcoreMesh(core_axis_name="c", subcore_axis_name="s")
    run = functools.partial(
        pl.kernel,
        out_type=jax.ShapeDtypeStruct((b,), jnp.float32),
        mesh=mesh,
        compiler_params=pltpu.CompilerParams(
            needs_layout_passes=False, use_tc_tiling_on_sc=False
        ),
        scratch_types=[
            pltpu.VMEM((bpw // IDX_CHUNK, IDX_CHUNK), jnp.int32),
            pltpu.VMEM((bpw // IDX_CHUNK, IDX_CHUNK), jnp.int32),
            pltpu.VMEM((bpw, f), jnp.float32),
            pltpu.VMEM((bpw, f), jnp.float32),
            pltpu.VMEM((f + LANES,), jnp.float32),
            pltpu.VMEM((bpw,), jnp.float32),
            pltpu.SemaphoreType.DMA,
        ],
    )(_gmf_body)
    return run(users_r, items_r, ut, it, wb)
